# baseline (speedup 1.0000x reference)
import functools

import jax
import jax.numpy as jnp
from jax import lax
from jax.experimental import pallas as pl
from jax.experimental.pallas import tpu as pltpu
from jax.experimental.pallas import tpu_sc as plsc

N_NODES = 10000
N_PAD = 10240
N_EDGES = 320000
CH = 128
N_GRAPHS = 16

NC = 2
NS = 16
NW = NC * NS
ECH = 128
EPT = 10240
NCHUNK = EPT // ECH
E_PAD = NW * EPT
STRIPE = N_PAD // NS

RBLK = 2048
NBLK = N_PAD // RBLK

def _mesh():
    return plsc.VectorSubcoreMesh(core_axis_name="c", subcore_axis_name="s",
                                  num_cores=NC, num_subcores=NS)


def _fill_rows(rows_v, value):
    def body(i, _):
        r = i // (CH // 16)
        c = i - r * (CH // 16)
        rows_v[r, pl.ds(c * 16, 16)] = jnp.full((16,), value, jnp.float32)
        return 0

    lax.fori_loop(0, ECH * (CH // 16), body, 0)


def _deg_body(em_hbm, out_hbm, acc, e2d, rows_v, sem):
    cid = lax.axis_index("c")
    sid = lax.axis_index("s")
    wid = cid * NS + sid

    pltpu.sync_copy(em_hbm.at[wid], e2d)
    _fill_rows(rows_v, 0.0)
    row0 = sid * STRIPE
    for k in range(STRIPE // ECH):
        pltpu.sync_copy(rows_v, acc.at[pl.ds(row0 + k * ECH, ECH)])
    plsc.subcore_barrier()

    _fill_rows(rows_v, 1.0)

    @pl.loop(0, NCHUNK, step=4)
    def _(j):
        descs = [pltpu.async_copy(rows_v, acc.at[e2d.at[j + b, 1]], sem,
                                  add=True) for b in range(4)]
        for d in descs:
            d.wait()

    plsc.subcore_barrier()
    pltpu.sync_copy(acc.at[pl.ds(row0, STRIPE)],
                    out_hbm.at[cid, pl.ds(row0, STRIPE)])


def _deg_kernel(edgem):
    return pl.kernel(
        _deg_body,
        out_type=jax.ShapeDtypeStruct((NC, N_PAD, CH), jnp.float32),
        mesh=_mesh(),
        scratch_types=[
            pltpu.VMEM_SHARED((N_PAD, CH), jnp.float32),
            pltpu.VMEM((NCHUNK, 2, ECH), jnp.int32),
            pltpu.VMEM((ECH, CH), jnp.float32),
            pltpu.SemaphoreType.DMA,
        ],
    )(edgem)


def _agg_body(h_hbm, em_hbm, out_hbm, acc,
              i0, i1, i2, i3, r0, r1,
              si0, si1, si2, si3, sr0, sr1, ss0, ss1):
    cid = lax.axis_index("c")
    sid = lax.axis_index("s")
    wid = cid * NS + sid

    _fill_rows(r0, 0.0)
    row0 = sid * STRIPE
    for k in range(STRIPE // ECH):
        pltpu.sync_copy(r0, acc.at[pl.ds(row0 + k * ECH, ECH)])
    plsc.subcore_barrier()

    ibufs = (i0, i1, i2, i3)
    rbufs = (r0, r1)
    isems = (si0, si1, si2, si3)
    rsems = (sr0, sr1)
    ssems = (ss0, ss1)
    pltpu.sync_copy(em_hbm.at[wid, 0], i0)
    pltpu.async_copy(em_hbm.at[wid, 1], i1, si1)
    pltpu.async_copy(em_hbm.at[wid, 2], i2, si2)
    pltpu.async_copy(h_hbm.at[i0.at[0]], r0, sr0)

    @pl.loop(0, NCHUNK, step=4)
    def _(j):
        for b in range(4):
            c = j + b
            nb = (b + 1) % 4
            fb = (b + 3) % 4

            pltpu.make_async_copy(h_hbm.at[ibufs[b].at[0]], rbufs[b % 2],
                                  rsems[b % 2]).wait()
            pltpu.async_copy(rbufs[b % 2], acc.at[ibufs[b].at[1]],
                             ssems[b % 2], add=True)

            @pl.when(c + 1 < NCHUNK)
            def _():
                pltpu.make_async_copy(em_hbm.at[wid, c + 1], ibufs[nb],
                                      isems[nb]).wait()

                @pl.when(c >= 1)
                def _():
                    pltpu.make_async_copy(
                        rbufs[nb % 2], acc.at[ibufs[fb].at[1]],
                        ssems[nb % 2]).wait()

                pltpu.async_copy(h_hbm.at[ibufs[nb].at[0]], rbufs[nb % 2],
                                 rsems[nb % 2])

            @pl.when(c + 3 < NCHUNK)
            def _():
                pltpu.async_copy(em_hbm.at[wid, c + 3], ibufs[fb], isems[fb])

    for b in range(2):
        pltpu.make_async_copy(rbufs[b], acc.at[i0.at[1]], ssems[b]).wait()
    plsc.subcore_barrier()
    pltpu.sync_copy(acc.at[pl.ds(row0, STRIPE)],
                    out_hbm.at[cid, pl.ds(row0, STRIPE)])


def _agg_kernel(h, edgem):
    return pl.kernel(
        _agg_body,
        out_type=jax.ShapeDtypeStruct((NC, N_PAD, CH), jnp.float32),
        mesh=_mesh(),
        scratch_types=[
            pltpu.VMEM_SHARED((N_PAD, CH), jnp.float32),
            pltpu.VMEM((2, ECH), jnp.int32),
            pltpu.VMEM((2, ECH), jnp.int32),
            pltpu.VMEM((2, ECH), jnp.int32),
            pltpu.VMEM((2, ECH), jnp.int32),
            pltpu.VMEM((ECH, CH), jnp.float32),
            pltpu.VMEM((ECH, CH), jnp.float32),
            pltpu.SemaphoreType.DMA,
            pltpu.SemaphoreType.DMA,
            pltpu.SemaphoreType.DMA,
            pltpu.SemaphoreType.DMA,
            pltpu.SemaphoreType.DMA,
            pltpu.SemaphoreType.DMA,
            pltpu.SemaphoreType.DMA,
            pltpu.SemaphoreType.DMA,
        ],
    )(h, edgem)


def _h1_body(x_ref, w_ref, dp_ref, o_ref, dis_ref):
    deg = dp_ref[0, :, 0:1] + dp_ref[1, :, 0:1] + 1.0
    dis = lax.rsqrt(deg)
    dis_ref[...] = jnp.broadcast_to(dis, dis_ref.shape)
    o_ref[...] = jnp.dot(x_ref[...], w_ref[...],
                         preferred_element_type=jnp.float32) * dis


def _stage_body(p_ref, hp_ref, dis16_ref, b_ref, w_ref, o_ref):
    dis = dis16_ref[:, 0:1]
    a = jnp.maximum((p_ref[0] + p_ref[1] + hp_ref[...]) * dis + b_ref[...],
                    0.0)
    o_ref[...] = jnp.dot(a, w_ref[...],
                         preferred_element_type=jnp.float32) * dis


def _pool_body(p_ref, hp_ref, dis16_ref, b_ref, batch_ref, wc_ref, bc_ref,
               o_ref, sums, counts):
    i = pl.program_id(0)

    @pl.when(i == 0)
    def _():
        sums[...] = jnp.zeros_like(sums)
        counts[...] = jnp.zeros_like(counts)

    dis = dis16_ref[:, 0:1]
    a = jnp.maximum((p_ref[0] + p_ref[1] + hp_ref[...]) * dis + b_ref[...],
                    0.0)
    bids = batch_ref[0]
    gids = lax.broadcasted_iota(jnp.int32, (N_GRAPHS, RBLK), 0)
    onehot = (bids == gids).astype(jnp.float32)
    sums[...] += jnp.dot(onehot, a, preferred_element_type=jnp.float32)
    counts[...] += jnp.sum(onehot, axis=1, keepdims=True)

    @pl.when(i == NBLK - 1)
    def _():
        pooled = sums[...] / jnp.maximum(counts[...], 1.0)
        o_ref[...] = jnp.dot(pooled, wc_ref[...],
                             preferred_element_type=jnp.float32) + bc_ref[...]


def _row_spec(ch):
    return pl.BlockSpec((RBLK, ch), lambda i: (i, 0))


def _part_spec(ch):
    return pl.BlockSpec((NC, RBLK, ch), lambda i: (0, i, 0))


_FULL_W = pl.BlockSpec((CH, CH), lambda i: (0, 0))
_FULL_B = pl.BlockSpec((1, CH), lambda i: (0, 0))


def _tc_h1(x, w, degp):
    return pl.pallas_call(
        _h1_body, grid=(NBLK,),
        in_specs=[_row_spec(CH), _FULL_W, _part_spec(CH)],
        out_specs=[_row_spec(CH), _row_spec(16)],
        out_shape=[jax.ShapeDtypeStruct((N_PAD, CH), jnp.float32),
                   jax.ShapeDtypeStruct((N_PAD, 16), jnp.float32)])(
            x, w, degp)


def _tc_stage(p, hp, dis16, b, w):
    return pl.pallas_call(
        _stage_body, grid=(NBLK,),
        in_specs=[_part_spec(CH), _row_spec(CH), _row_spec(16),
                  _FULL_B, _FULL_W],
        out_specs=_row_spec(CH),
        out_shape=jax.ShapeDtypeStruct((N_PAD, CH), jnp.float32))(
            p, hp, dis16, b.reshape(1, CH), w)


def _tc_pool(p, hp, dis16, b, batch3, wc_pad, bc_pad):
    return pl.pallas_call(
        _pool_body, grid=(NBLK,),
        in_specs=[_part_spec(CH), _row_spec(CH), _row_spec(16), _FULL_B,
                  pl.BlockSpec((1, 1, RBLK), lambda i: (i, 0, 0)),
                  _FULL_W, _FULL_B],
        out_specs=pl.BlockSpec((N_GRAPHS, CH), lambda i: (0, 0)),
        out_shape=jax.ShapeDtypeStruct((N_GRAPHS, CH), jnp.float32),
        scratch_shapes=[pltpu.VMEM((N_GRAPHS, CH), jnp.float32),
                        pltpu.VMEM((N_GRAPHS, CH), jnp.float32)])(
            p, hp, dis16, b.reshape(1, CH), batch3, wc_pad, bc_pad)


def kernel(x, edge_index, batch, W1, b1, W2, b2, W3, b3, Wc, bc):
    src = edge_index[0].astype(jnp.int32)
    dst = edge_index[1].astype(jnp.int32)
    npad_e = E_PAD - N_EDGES
    pad_ids = jnp.arange(npad_e, dtype=jnp.int32)
    srcp = jnp.concatenate([src, pad_ids % N_NODES])
    dstp = jnp.concatenate([dst, N_NODES + pad_ids % (N_PAD - N_NODES)])
    edgem = jnp.stack([srcp.reshape(NW, NCHUNK, ECH),
                       dstp.reshape(NW, NCHUNK, ECH)], axis=2)
    x_pad = jnp.pad(x, ((0, N_PAD - N_NODES), (0, 0)))
    batch3 = jnp.pad(batch.astype(jnp.int32), (0, N_PAD - N_NODES),
                     constant_values=N_GRAPHS).reshape(NBLK, 1, RBLK)
    wc_pad = jnp.pad(Wc, ((0, 0), (0, CH - Wc.shape[1])))
    bc_pad = jnp.pad(bc, (0, CH - bc.shape[0])).reshape(1, CH)

    degp = _deg_kernel(edgem)
    h1p, dis16 = _tc_h1(x_pad, W1, degp)
    p1 = _agg_kernel(h1p, edgem)
    h2p = _tc_stage(p1, h1p, dis16, b1, W2)
    p2 = _agg_kernel(h2p, edgem)
    h3p = _tc_stage(p2, h2p, dis16, b2, W3)
    p3 = _agg_kernel(h3p, edgem)
    out_pad = _tc_pool(p3, h3p, dis16, b3, batch3, wc_pad, bc_pad)
    return out_pad[:, :Wc.shape[1]]


# speedup vs baseline: 1.1562x; 1.1562x over previous
import functools

import jax
import jax.numpy as jnp
from jax import lax
from jax.experimental import pallas as pl
from jax.experimental.pallas import tpu as pltpu
from jax.experimental.pallas import tpu_sc as plsc

N_NODES = 10000
N_PAD = 10240
N_EDGES = 320000
CH = 128
N_GRAPHS = 16

NC = 2
NS = 16
NW = NC * NS
ECH = 128
EPT = 10240
NCHUNK = EPT // ECH
E_PAD = NW * EPT
STRIPE = N_PAD // NS

RBLK = 2048
NBLK = N_PAD // RBLK

def _mesh():
    return plsc.VectorSubcoreMesh(core_axis_name="c", subcore_axis_name="s",
                                  num_cores=NC, num_subcores=NS)


def _fill_rows(rows_v, value):
    def body(i, _):
        r = i // (CH // 16)
        c = i - r * (CH // 16)
        rows_v[r, pl.ds(c * 16, 16)] = jnp.full((16,), value, jnp.float32)
        return 0

    lax.fori_loop(0, ECH * (CH // 16), body, 0)


def _deg_body(em_hbm, out_hbm, acc, e2d, rows_v, sem):
    cid = lax.axis_index("c")
    sid = lax.axis_index("s")
    wid = cid * NS + sid

    pltpu.sync_copy(em_hbm.at[wid], e2d)
    _fill_rows(rows_v, 0.0)
    row0 = sid * STRIPE
    for k in range(STRIPE // ECH):
        pltpu.sync_copy(rows_v, acc.at[pl.ds(row0 + k * ECH, ECH)])
    plsc.subcore_barrier()

    _fill_rows(rows_v, 1.0)

    @pl.loop(0, NCHUNK, step=4)
    def _(j):
        descs = [pltpu.async_copy(rows_v, acc.at[e2d.at[j + b, 1]], sem,
                                  add=True) for b in range(4)]
        for d in descs:
            d.wait()

    plsc.subcore_barrier()
    pltpu.sync_copy(acc.at[pl.ds(row0, STRIPE)],
                    out_hbm.at[cid, pl.ds(row0, STRIPE)])


def _deg_kernel(edgem):
    return pl.kernel(
        _deg_body,
        out_type=jax.ShapeDtypeStruct((NC, N_PAD, CH), jnp.float32),
        mesh=_mesh(),
        scratch_types=[
            pltpu.VMEM_SHARED((N_PAD, CH), jnp.float32),
            pltpu.VMEM((NCHUNK, 2, ECH), jnp.int32),
            pltpu.VMEM((ECH, CH), jnp.float32),
            pltpu.SemaphoreType.DMA,
        ],
    )(edgem)


def _agg_body(h_hbm, em_hbm, out_hbm, acc,
              i0, i1, i2, i3, r0, r1,
              si0, si1, si2, si3, sr0, sr1):
    cid = lax.axis_index("c")
    sid = lax.axis_index("s")
    wid = cid * NS + sid

    _fill_rows(r0, 0.0)
    row0 = sid * STRIPE
    for k in range(STRIPE // ECH):
        pltpu.sync_copy(r0, acc.at[pl.ds(row0 + k * ECH, ECH)])
    plsc.subcore_barrier()

    ibufs = (i0, i1, i2, i3)
    rbufs = (r0, r1)
    isems = (si0, si1, si2, si3)
    rsems = (sr0, sr1)
    pltpu.sync_copy(em_hbm.at[wid, 0], i0)
    pltpu.async_copy(em_hbm.at[wid, 1], i1, si1)
    pltpu.async_copy(em_hbm.at[wid, 2], i2, si2)
    pltpu.async_copy(h_hbm.at[i0.at[0]], r0, sr0)

    @pl.loop(0, NCHUNK, step=4)
    def _(j):
        for b in range(4):
            c = j + b
            nb = (b + 1) % 4
            fb = (b + 3) % 4

            @pl.when(c + 1 < NCHUNK)
            def _():
                pltpu.make_async_copy(em_hbm.at[wid, c + 1], ibufs[nb],
                                      isems[nb]).wait()
                pltpu.async_copy(h_hbm.at[ibufs[nb].at[0]], rbufs[nb % 2],
                                 rsems[nb % 2])

            pltpu.make_async_copy(h_hbm.at[ibufs[b].at[0]], rbufs[b % 2],
                                  rsems[b % 2]).wait()
            pltpu.sync_copy(rbufs[b % 2], acc.at[ibufs[b].at[1]], add=True)

            @pl.when(c + 3 < NCHUNK)
            def _():
                pltpu.async_copy(em_hbm.at[wid, c + 3], ibufs[fb], isems[fb])

    plsc.subcore_barrier()
    pltpu.sync_copy(acc.at[pl.ds(row0, STRIPE)],
                    out_hbm.at[cid, pl.ds(row0, STRIPE)])


def _agg_kernel(h, edgem):
    return pl.kernel(
        _agg_body,
        out_type=jax.ShapeDtypeStruct((NC, N_PAD, CH), jnp.float32),
        mesh=_mesh(),
        scratch_types=[
            pltpu.VMEM_SHARED((N_PAD, CH), jnp.float32),
            pltpu.VMEM((2, ECH), jnp.int32),
            pltpu.VMEM((2, ECH), jnp.int32),
            pltpu.VMEM((2, ECH), jnp.int32),
            pltpu.VMEM((2, ECH), jnp.int32),
            pltpu.VMEM((ECH, CH), jnp.float32),
            pltpu.VMEM((ECH, CH), jnp.float32),
            pltpu.SemaphoreType.DMA,
            pltpu.SemaphoreType.DMA,
            pltpu.SemaphoreType.DMA,
            pltpu.SemaphoreType.DMA,
            pltpu.SemaphoreType.DMA,
            pltpu.SemaphoreType.DMA,
        ],
    )(h, edgem)


def _h1_body(x_ref, w_ref, dp_ref, o_ref, dis_ref):
    deg = dp_ref[0, :, 0:1] + dp_ref[1, :, 0:1] + 1.0
    dis = lax.rsqrt(deg)
    dis_ref[...] = jnp.broadcast_to(dis, dis_ref.shape)
    o_ref[...] = jnp.dot(x_ref[...], w_ref[...],
                         preferred_element_type=jnp.float32) * dis


def _stage_body(p_ref, hp_ref, dis16_ref, b_ref, w_ref, o_ref):
    dis = dis16_ref[:, 0:1]
    a = jnp.maximum((p_ref[0] + p_ref[1] + hp_ref[...]) * dis + b_ref[...],
                    0.0)
    o_ref[...] = jnp.dot(a, w_ref[...],
                         preferred_element_type=jnp.float32) * dis


def _pool_body(p_ref, hp_ref, dis16_ref, b_ref, batch_ref, wc_ref, bc_ref,
               o_ref, sums, counts):
    i = pl.program_id(0)

    @pl.when(i == 0)
    def _():
        sums[...] = jnp.zeros_like(sums)
        counts[...] = jnp.zeros_like(counts)

    dis = dis16_ref[:, 0:1]
    a = jnp.maximum((p_ref[0] + p_ref[1] + hp_ref[...]) * dis + b_ref[...],
                    0.0)
    bids = batch_ref[0]
    gids = lax.broadcasted_iota(jnp.int32, (N_GRAPHS, RBLK), 0)
    onehot = (bids == gids).astype(jnp.float32)
    sums[...] += jnp.dot(onehot, a, preferred_element_type=jnp.float32)
    counts[...] += jnp.sum(onehot, axis=1, keepdims=True)

    @pl.when(i == NBLK - 1)
    def _():
        pooled = sums[...] / jnp.maximum(counts[...], 1.0)
        o_ref[...] = jnp.dot(pooled, wc_ref[...],
                             preferred_element_type=jnp.float32) + bc_ref[...]


def _row_spec(ch):
    return pl.BlockSpec((RBLK, ch), lambda i: (i, 0))


def _part_spec(ch):
    return pl.BlockSpec((NC, RBLK, ch), lambda i: (0, i, 0))


_FULL_W = pl.BlockSpec((CH, CH), lambda i: (0, 0))
_FULL_B = pl.BlockSpec((1, CH), lambda i: (0, 0))


def _tc_h1(x, w, degp):
    return pl.pallas_call(
        _h1_body, grid=(NBLK,),
        in_specs=[_row_spec(CH), _FULL_W, _part_spec(CH)],
        out_specs=[_row_spec(CH), _row_spec(16)],
        out_shape=[jax.ShapeDtypeStruct((N_PAD, CH), jnp.float32),
                   jax.ShapeDtypeStruct((N_PAD, 16), jnp.float32)])(
            x, w, degp)


def _tc_stage(p, hp, dis16, b, w):
    return pl.pallas_call(
        _stage_body, grid=(NBLK,),
        in_specs=[_part_spec(CH), _row_spec(CH), _row_spec(16),
                  _FULL_B, _FULL_W],
        out_specs=_row_spec(CH),
        out_shape=jax.ShapeDtypeStruct((N_PAD, CH), jnp.float32))(
            p, hp, dis16, b.reshape(1, CH), w)


def _tc_pool(p, hp, dis16, b, batch3, wc_pad, bc_pad):
    return pl.pallas_call(
        _pool_body, grid=(NBLK,),
        in_specs=[_part_spec(CH), _row_spec(CH), _row_spec(16), _FULL_B,
                  pl.BlockSpec((1, 1, RBLK), lambda i: (i, 0, 0)),
                  _FULL_W, _FULL_B],
        out_specs=pl.BlockSpec((N_GRAPHS, CH), lambda i: (0, 0)),
        out_shape=jax.ShapeDtypeStruct((N_GRAPHS, CH), jnp.float32),
        scratch_shapes=[pltpu.VMEM((N_GRAPHS, CH), jnp.float32),
                        pltpu.VMEM((N_GRAPHS, CH), jnp.float32)])(
            p, hp, dis16, b.reshape(1, CH), batch3, wc_pad, bc_pad)


def kernel(x, edge_index, batch, W1, b1, W2, b2, W3, b3, Wc, bc):
    src = edge_index[0].astype(jnp.int32)
    dst = edge_index[1].astype(jnp.int32)
    npad_e = E_PAD - N_EDGES
    pad_ids = jnp.arange(npad_e, dtype=jnp.int32)
    srcp = jnp.concatenate([src, pad_ids % N_NODES])
    dstp = jnp.concatenate([dst, N_NODES + pad_ids % (N_PAD - N_NODES)])
    edgem = jnp.stack([srcp.reshape(NW, NCHUNK, ECH),
                       dstp.reshape(NW, NCHUNK, ECH)], axis=2)
    x_pad = jnp.pad(x, ((0, N_PAD - N_NODES), (0, 0)))
    batch3 = jnp.pad(batch.astype(jnp.int32), (0, N_PAD - N_NODES),
                     constant_values=N_GRAPHS).reshape(NBLK, 1, RBLK)
    wc_pad = jnp.pad(Wc, ((0, 0), (0, CH - Wc.shape[1])))
    bc_pad = jnp.pad(bc, (0, CH - bc.shape[0])).reshape(1, CH)

    degp = _deg_kernel(edgem)
    h1p, dis16 = _tc_h1(x_pad, W1, degp)
    p1 = _agg_kernel(h1p, edgem)
    h2p = _tc_stage(p1, h1p, dis16, b1, W2)
    p2 = _agg_kernel(h2p, edgem)
    h3p = _tc_stage(p2, h2p, dis16, b2, W3)
    p3 = _agg_kernel(h3p, edgem)
    out_pad = _tc_pool(p3, h3p, dis16, b3, batch3, wc_pad, bc_pad)
    return out_pad[:, :Wc.shape[1]]
